# Initial kernel scaffold; baseline (speedup 1.0000x reference)
import jax
import jax.numpy as jnp
from jax.experimental import pallas as pl


def kernel(x, conv1_w, bn1_scale, bn1_bias, L0b0_conv1_w, L0b0_bn1_scale, L0b0_bn1_bias, L0b0_conv2_w, L0b0_bn2_scale, L0b0_bn2_bias, L0b1_conv1_w, L0b1_bn1_scale, L0b1_bn1_bias, L0b1_conv2_w, L0b1_bn2_scale, L0b1_bn2_bias, L0b2_conv1_w, L0b2_bn1_scale, L0b2_bn1_bias, L0b2_conv2_w, L0b2_bn2_scale, L0b2_bn2_bias, L1b0_conv1_w, L1b0_bn1_scale, L1b0_bn1_bias, L1b0_conv2_w, L1b0_bn2_scale, L1b0_bn2_bias, L1b0_down_w, L1b0_down_scale, L1b0_down_bias, L1b1_conv1_w, L1b1_bn1_scale, L1b1_bn1_bias, L1b1_conv2_w, L1b1_bn2_scale, L1b1_bn2_bias, L1b2_conv1_w, L1b2_bn1_scale, L1b2_bn1_bias, L1b2_conv2_w, L1b2_bn2_scale, L1b2_bn2_bias, L1b3_conv1_w, L1b3_bn1_scale, L1b3_bn1_bias, L1b3_conv2_w, L1b3_bn2_scale, L1b3_bn2_bias, L2b0_conv1_w, L2b0_bn1_scale, L2b0_bn1_bias, L2b0_conv2_w, L2b0_bn2_scale, L2b0_bn2_bias, L2b0_down_w, L2b0_down_scale, L2b0_down_bias, L2b1_conv1_w, L2b1_bn1_scale, L2b1_bn1_bias, L2b1_conv2_w, L2b1_bn2_scale, L2b1_bn2_bias, L2b2_conv1_w, L2b2_bn1_scale, L2b2_bn1_bias, L2b2_conv2_w, L2b2_bn2_scale, L2b2_bn2_bias, L2b3_conv1_w, L2b3_bn1_scale, L2b3_bn1_bias, L2b3_conv2_w, L2b3_bn2_scale, L2b3_bn2_bias, L2b4_conv1_w, L2b4_bn1_scale, L2b4_bn1_bias, L2b4_conv2_w, L2b4_bn2_scale, L2b4_bn2_bias, L2b5_conv1_w, L2b5_bn1_scale, L2b5_bn1_bias, L2b5_conv2_w, L2b5_bn2_scale, L2b5_bn2_bias, L3b0_conv1_w, L3b0_bn1_scale, L3b0_bn1_bias, L3b0_conv2_w, L3b0_bn2_scale, L3b0_bn2_bias, L3b0_down_w, L3b0_down_scale, L3b0_down_bias, L3b1_conv1_w, L3b1_bn1_scale, L3b1_bn1_bias, L3b1_conv2_w, L3b1_bn2_scale, L3b1_bn2_bias, L3b2_conv1_w, L3b2_bn1_scale, L3b2_bn1_bias, L3b2_conv2_w, L3b2_bn2_scale, L3b2_bn2_bias, fc_w, fc_b):
    raise NotImplementedError("write your pallas kernel here")


# speedup vs baseline: 1.1043x; 1.1043x over previous
import functools
import itertools

import jax
import jax.numpy as jnp
from jax.experimental import pallas as pl
from jax.experimental.pallas import tpu as pltpu

_DT = jnp.bfloat16
_VMEM = 48 * 1024 * 1024
_TAPS3 = tuple(itertools.product(range(3), range(3)))


def _mask_cols(y, Wp, W):
    col = jax.lax.broadcasted_iota(jnp.int32, y.shape, 0) % Wp
    return jnp.where(col < W, y, 0.0)


def _fused_block_kernel(x_ref, w1_ref, s1_ref, b1_ref, w2_ref, s2_ref, b2_ref,
                        o_ref, acc_ref, mid_ref, *, H, W):
    Wp = W + 2
    l_out = H * Wp
    off0 = Wp + 1
    C = x_ref.shape[1]
    zrow = jnp.zeros((off0, C), _DT)
    mid_ref[0:off0, :] = zrow
    mid_ref[off0 + l_out:, :] = zrow
    for t, (di, dj) in enumerate(_TAPS3):
        off = di * Wp + dj
        c = jnp.dot(x_ref[off:off + l_out, :], w1_ref[t],
                    preferred_element_type=jnp.float32)
        if t == 0:
            acc_ref[...] = c
        else:
            acc_ref[...] += c
    y = jnp.maximum(acc_ref[...] * s1_ref[...] + b1_ref[...], 0.0)
    y = _mask_cols(y, Wp, W)
    mid_ref[off0:off0 + l_out, :] = y.astype(_DT)
    for t, (di, dj) in enumerate(_TAPS3):
        off = di * Wp + dj
        c = jnp.dot(mid_ref[off:off + l_out, :], w2_ref[t],
                    preferred_element_type=jnp.float32)
        if t == 0:
            acc_ref[...] = c
        else:
            acc_ref[...] += c
    y = acc_ref[...] * s2_ref[...] + b2_ref[...] \
        + x_ref[off0:off0 + l_out, :].astype(jnp.float32)
    y = jnp.maximum(y, 0.0)
    y = _mask_cols(y, Wp, W)
    o_ref[0:off0, :] = zrow
    o_ref[off0:off0 + l_out, :] = y.astype(_DT)
    o_ref[off0 + l_out:, :] = zrow


def _fused_block(xpad, w1, s1, b1, w2, s2, b2, H, W, C):
    N = xpad.shape[0]
    L = (H + 2) * (W + 2) + 2
    l_out = H * (W + 2)
    return pl.pallas_call(
        functools.partial(_fused_block_kernel, H=H, W=W),
        out_shape=jax.ShapeDtypeStruct((N, L, C), _DT),
        grid=(N,),
        in_specs=[
            pl.BlockSpec((None, L, C), lambda n: (n, 0, 0)),
            pl.BlockSpec((9, C, C), lambda n: (0, 0, 0)),
            pl.BlockSpec((1, C), lambda n: (0, 0)),
            pl.BlockSpec((1, C), lambda n: (0, 0)),
            pl.BlockSpec((9, C, C), lambda n: (0, 0, 0)),
            pl.BlockSpec((1, C), lambda n: (0, 0)),
            pl.BlockSpec((1, C), lambda n: (0, 0)),
        ],
        out_specs=pl.BlockSpec((None, L, C), lambda n: (n, 0, 0)),
        scratch_shapes=[pltpu.VMEM((l_out, C), jnp.float32),
                        pltpu.VMEM((L, C), _DT)],
        compiler_params=pltpu.CompilerParams(
            dimension_semantics=("parallel",),
            vmem_limit_bytes=_VMEM),
    )(xpad, w1, s1, b1, w2, s2, b2)


def _down_pair_kernel(x_ref, w1_ref, s1_ref, b1_ref, wd_ref, sd_ref, bd_ref,
                      y_ref, r_ref, acc_ref, *, taps, l_out):
    for t, (p, off) in enumerate(taps):
        c = jnp.dot(x_ref[p, off:off + l_out, :], w1_ref[t],
                    preferred_element_type=jnp.float32)
        if t == 0:
            acc_ref[...] = c
        else:
            acc_ref[...] += c
    y = jnp.maximum(acc_ref[...] * s1_ref[...] + b1_ref[...], 0.0)
    y_ref[...] = y.astype(_DT)
    rd = jnp.dot(x_ref[3, 0:l_out, :], wd_ref[0],
                 preferred_element_type=jnp.float32)
    r_ref[...] = (rd * sd_ref[...] + bd_ref[...]).astype(_DT)


def _conv_res_pad_kernel(x_ref, w_ref, s_ref, b_ref, r_ref, o_ref, acc_ref,
                         *, H, W):
    Wp = W + 2
    l_out = H * Wp
    off0 = Wp + 1
    C = o_ref.shape[1]
    for t, (di, dj) in enumerate(_TAPS3):
        off = di * Wp + dj
        c = jnp.dot(x_ref[off:off + l_out, :], w_ref[t],
                    preferred_element_type=jnp.float32)
        if t == 0:
            acc_ref[...] = c
        else:
            acc_ref[...] += c
    y = acc_ref[...] * s_ref[...] + b_ref[...] + r_ref[...].astype(jnp.float32)
    y = jnp.maximum(y, 0.0)
    y = _mask_cols(y, Wp, W)
    zrow = jnp.zeros((off0, C), _DT)
    o_ref[0:off0, :] = zrow
    o_ref[off0:off0 + l_out, :] = y.astype(_DT)
    o_ref[off0 + l_out:, :] = zrow


def _to_padded(x):
    N, H, W, C = x.shape
    xp = jnp.pad(x, ((0, 0), (1, 1), (1, 1), (0, 0)))
    return jnp.pad(xp.reshape(N, (H + 2) * (W + 2), C), ((0, 0), (0, 2), (0, 0)))


def _down_block(xpad, H, W, Cin, Cout, w1, s1, b1, w2, s2, b2, wd, sd, bd):
    N = xpad.shape[0]
    Wp = W + 2
    OH, OW = H // 2, W // 2
    PH, PW = OH + 1, OW + 1
    l_in = PH * PW + 1
    l_out1 = OH * PW
    xp = xpad[:, :(H + 2) * Wp, :].reshape(N, H + 2, Wp, Cin)
    phs = []
    for a in (0, 1):
        for b_ in (0, 1):
            sub = xp[:, a::2, b_::2, :][:, :PH, :PW, :]
            phs.append(sub.reshape(N, PH * PW, Cin))
    ph = jnp.stack(phs, 0)
    ph = jnp.pad(ph, ((0, 0), (0, 0), (0, l_in - PH * PW), (0, 0)))
    taps = [((ki % 2) * 2 + (kj % 2), (ki // 2) * PW + (kj // 2))
            for ki, kj in _TAPS3]
    y1, r = pl.pallas_call(
        functools.partial(_down_pair_kernel, taps=taps, l_out=l_out1),
        out_shape=[jax.ShapeDtypeStruct((N, l_out1, Cout), _DT),
                   jax.ShapeDtypeStruct((N, l_out1, Cout), _DT)],
        grid=(N,),
        in_specs=[
            pl.BlockSpec((4, None, l_in, Cin), lambda n: (0, n, 0, 0)),
            pl.BlockSpec((9, Cin, Cout), lambda n: (0, 0, 0)),
            pl.BlockSpec((1, Cout), lambda n: (0, 0)),
            pl.BlockSpec((1, Cout), lambda n: (0, 0)),
            pl.BlockSpec((1, Cin, Cout), lambda n: (0, 0, 0)),
            pl.BlockSpec((1, Cout), lambda n: (0, 0)),
            pl.BlockSpec((1, Cout), lambda n: (0, 0)),
        ],
        out_specs=[pl.BlockSpec((None, l_out1, Cout), lambda n: (n, 0, 0)),
                   pl.BlockSpec((None, l_out1, Cout), lambda n: (n, 0, 0))],
        scratch_shapes=[pltpu.VMEM((l_out1, Cout), jnp.float32)],
        compiler_params=pltpu.CompilerParams(
            dimension_semantics=("parallel",),
            vmem_limit_bytes=_VMEM),
    )(ph, w1, s1, b1, wd, sd, bd)
    y1p = _to_padded(y1.reshape(N, OH, PW, Cout)[:, :, :OW, :])
    rr = jnp.pad(r.reshape(N, OH, PW, Cout)[:, :, :OW, :],
                 ((0, 0), (0, 0), (0, 2), (0, 0)))
    rr = rr.reshape(N, OH * (OW + 2), Cout)
    L2 = (OH + 2) * (OW + 2) + 2
    l_out2 = OH * (OW + 2)
    return pl.pallas_call(
        functools.partial(_conv_res_pad_kernel, H=OH, W=OW),
        out_shape=jax.ShapeDtypeStruct((N, L2, Cout), _DT),
        grid=(N,),
        in_specs=[
            pl.BlockSpec((None, L2, Cout), lambda n: (n, 0, 0)),
            pl.BlockSpec((9, Cout, Cout), lambda n: (0, 0, 0)),
            pl.BlockSpec((1, Cout), lambda n: (0, 0)),
            pl.BlockSpec((1, Cout), lambda n: (0, 0)),
            pl.BlockSpec((None, l_out2, Cout), lambda n: (n, 0, 0)),
        ],
        out_specs=pl.BlockSpec((None, L2, Cout), lambda n: (n, 0, 0)),
        scratch_shapes=[pltpu.VMEM((l_out2, Cout), jnp.float32)],
        compiler_params=pltpu.CompilerParams(
            dimension_semantics=("parallel",),
            vmem_limit_bytes=_VMEM),
    )(y1p, w2, s2, b2, rr)


def _stem_kernel(x_ref, w_ref, s_ref, b_ref, o_ref, acc_ref, *, taps, l_out):
    for t, (p, off) in enumerate(taps):
        c = jnp.dot(x_ref[p, off:off + l_out, :], w_ref[t],
                    preferred_element_type=jnp.float32)
        if t == 0:
            acc_ref[...] = c
        else:
            acc_ref[...] += c
    o_ref[...] = (acc_ref[...] * s_ref[...] + b_ref[...]).astype(_DT)


def _maxpool_kernel(x_ref, o_ref, *, taps, l_out):
    p0, off0 = taps[0]
    acc = x_ref[p0, off0:off0 + l_out, :]
    for (p, off) in taps[1:]:
        acc = jnp.maximum(acc, x_ref[p, off:off + l_out, :])
    o_ref[...] = acc


def _head_kernel(x_ref, w_ref, b_ref, o_ref, *, inv_hw):
    pooled = jnp.sum(x_ref[...].astype(jnp.float32), axis=1) * inv_hw
    y = jnp.dot(pooled.astype(_DT), w_ref[...],
                preferred_element_type=jnp.float32)
    o_ref[...] = y + b_ref[...]


def _build_phases_s2(x, pad, PH, PW, l_in, pad_value):
    N, H, W, C = x.shape
    xp = jnp.pad(x, ((0, 0), (pad, pad), (pad, pad), (0, 0)),
                 constant_values=pad_value)
    phs = []
    for a in (0, 1):
        for b in (0, 1):
            sub = xp[:, a::2, b::2, :][:, :PH, :PW, :]
            ph_pad = PH - sub.shape[1]
            pw_pad = PW - sub.shape[2]
            if ph_pad or pw_pad:
                sub = jnp.pad(sub, ((0, 0), (0, ph_pad), (0, pw_pad), (0, 0)),
                              constant_values=pad_value)
            phs.append(sub.reshape(N, PH * PW, C))
    st = jnp.stack(phs, 0)
    return jnp.pad(st, ((0, 0), (0, 0), (0, l_in - PH * PW), (0, 0)),
                   constant_values=pad_value)


def kernel(x, conv1_w, bn1_scale, bn1_bias, L0b0_conv1_w, L0b0_bn1_scale, L0b0_bn1_bias, L0b0_conv2_w, L0b0_bn2_scale, L0b0_bn2_bias, L0b1_conv1_w, L0b1_bn1_scale, L0b1_bn1_bias, L0b1_conv2_w, L0b1_bn2_scale, L0b1_bn2_bias, L0b2_conv1_w, L0b2_bn1_scale, L0b2_bn1_bias, L0b2_conv2_w, L0b2_bn2_scale, L0b2_bn2_bias, L1b0_conv1_w, L1b0_bn1_scale, L1b0_bn1_bias, L1b0_conv2_w, L1b0_bn2_scale, L1b0_bn2_bias, L1b0_down_w, L1b0_down_scale, L1b0_down_bias, L1b1_conv1_w, L1b1_bn1_scale, L1b1_bn1_bias, L1b1_conv2_w, L1b1_bn2_scale, L1b1_bn2_bias, L1b2_conv1_w, L1b2_bn1_scale, L1b2_bn1_bias, L1b2_conv2_w, L1b2_bn2_scale, L1b2_bn2_bias, L1b3_conv1_w, L1b3_bn1_scale, L1b3_bn1_bias, L1b3_conv2_w, L1b3_bn2_scale, L1b3_bn2_bias, L2b0_conv1_w, L2b0_bn1_scale, L2b0_bn1_bias, L2b0_conv2_w, L2b0_bn2_scale, L2b0_bn2_bias, L2b0_down_w, L2b0_down_scale, L2b0_down_bias, L2b1_conv1_w, L2b1_bn1_scale, L2b1_bn1_bias, L2b1_conv2_w, L2b1_bn2_scale, L2b1_bn2_bias, L2b2_conv1_w, L2b2_bn1_scale, L2b2_bn1_bias, L2b2_conv2_w, L2b2_bn2_scale, L2b2_bn2_bias, L2b3_conv1_w, L2b3_bn1_scale, L2b3_bn1_bias, L2b3_conv2_w, L2b3_bn2_scale, L2b3_bn2_bias, L2b4_conv1_w, L2b4_bn1_scale, L2b4_bn1_bias, L2b4_conv2_w, L2b4_bn2_scale, L2b4_bn2_bias, L2b5_conv1_w, L2b5_bn1_scale, L2b5_bn1_bias, L2b5_conv2_w, L2b5_bn2_scale, L2b5_bn2_bias, L3b0_conv1_w, L3b0_bn1_scale, L3b0_bn1_bias, L3b0_conv2_w, L3b0_bn2_scale, L3b0_bn2_bias, L3b0_down_w, L3b0_down_scale, L3b0_down_bias, L3b1_conv1_w, L3b1_bn1_scale, L3b1_bn1_bias, L3b1_conv2_w, L3b1_bn2_scale, L3b1_bn2_bias, L3b2_conv1_w, L3b2_bn1_scale, L3b2_bn1_bias, L3b2_conv2_w, L3b2_bn2_scale, L3b2_bn2_bias, fc_w, fc_b):
    N = x.shape[0]
    xh = jnp.transpose(x, (0, 2, 3, 1)).astype(_DT)

    OH, PW = 112, 115
    l_in = 115 * 115 + 3
    l_out = 112 * 115
    ph = _build_phases_s2(xh, 3, 115, 115, l_in, 0.0)
    taps = [((ki % 2) * 2 + (kj % 2), (ki // 2) * 115 + (kj // 2))
            for ki, kj in itertools.product(range(7), range(7))]
    y = pl.pallas_call(
        functools.partial(_stem_kernel, taps=taps, l_out=l_out),
        out_shape=jax.ShapeDtypeStruct((N, l_out, 64), _DT),
        grid=(N,),
        in_specs=[
            pl.BlockSpec((4, None, l_in, 3), lambda n: (0, n, 0, 0)),
            pl.BlockSpec((49, 3, 64), lambda n: (0, 0, 0)),
            pl.BlockSpec((1, 64), lambda n: (0, 0)),
            pl.BlockSpec((1, 64), lambda n: (0, 0)),
        ],
        out_specs=pl.BlockSpec((None, l_out, 64), lambda n: (n, 0, 0)),
        scratch_shapes=[pltpu.VMEM((l_out, 64), jnp.float32)],
        compiler_params=pltpu.CompilerParams(
            dimension_semantics=("parallel",),
            vmem_limit_bytes=_VMEM),
    )(ph, conv1_w, bn1_scale, bn1_bias)
    y = y.reshape(N, 112, 115, 64)[:, :, :112, :]

    l_in = 57 * 57 + 1
    l_out = 56 * 57
    ph = _build_phases_s2(y, 1, 57, 57, l_in, -jnp.inf)
    taps = [((ki % 2) * 2 + (kj % 2), (ki // 2) * 57 + (kj // 2))
            for ki, kj in _TAPS3]
    y = pl.pallas_call(
        functools.partial(_maxpool_kernel, taps=taps, l_out=l_out),
        out_shape=jax.ShapeDtypeStruct((N, l_out, 64), _DT),
        grid=(N,),
        in_specs=[pl.BlockSpec((4, None, l_in, 64), lambda n: (0, n, 0, 0))],
        out_specs=pl.BlockSpec((None, l_out, 64), lambda n: (n, 0, 0)),
        compiler_params=pltpu.CompilerParams(
            dimension_semantics=("parallel",),
            vmem_limit_bytes=_VMEM),
    )(ph)
    y = y.reshape(N, 56, 57, 64)[:, :, :56, :]

    xp = _to_padded(y)

    for (w1, s1, b1, w2, s2, b2) in (
            (L0b0_conv1_w, L0b0_bn1_scale, L0b0_bn1_bias,
             L0b0_conv2_w, L0b0_bn2_scale, L0b0_bn2_bias),
            (L0b1_conv1_w, L0b1_bn1_scale, L0b1_bn1_bias,
             L0b1_conv2_w, L0b1_bn2_scale, L0b1_bn2_bias),
            (L0b2_conv1_w, L0b2_bn1_scale, L0b2_bn1_bias,
             L0b2_conv2_w, L0b2_bn2_scale, L0b2_bn2_bias)):
        xp = _fused_block(xp, w1, s1, b1, w2, s2, b2, 56, 56, 64)

    xp = _down_block(xp, 56, 56, 64, 128,
                     L1b0_conv1_w, L1b0_bn1_scale, L1b0_bn1_bias,
                     L1b0_conv2_w, L1b0_bn2_scale, L1b0_bn2_bias,
                     L1b0_down_w, L1b0_down_scale, L1b0_down_bias)
    for (w1, s1, b1, w2, s2, b2) in (
            (L1b1_conv1_w, L1b1_bn1_scale, L1b1_bn1_bias,
             L1b1_conv2_w, L1b1_bn2_scale, L1b1_bn2_bias),
            (L1b2_conv1_w, L1b2_bn1_scale, L1b2_bn1_bias,
             L1b2_conv2_w, L1b2_bn2_scale, L1b2_bn2_bias),
            (L1b3_conv1_w, L1b3_bn1_scale, L1b3_bn1_bias,
             L1b3_conv2_w, L1b3_bn2_scale, L1b3_bn2_bias)):
        xp = _fused_block(xp, w1, s1, b1, w2, s2, b2, 28, 28, 128)

    xp = _down_block(xp, 28, 28, 128, 256,
                     L2b0_conv1_w, L2b0_bn1_scale, L2b0_bn1_bias,
                     L2b0_conv2_w, L2b0_bn2_scale, L2b0_bn2_bias,
                     L2b0_down_w, L2b0_down_scale, L2b0_down_bias)
    for (w1, s1, b1, w2, s2, b2) in (
            (L2b1_conv1_w, L2b1_bn1_scale, L2b1_bn1_bias,
             L2b1_conv2_w, L2b1_bn2_scale, L2b1_bn2_bias),
            (L2b2_conv1_w, L2b2_bn1_scale, L2b2_bn1_bias,
             L2b2_conv2_w, L2b2_bn2_scale, L2b2_bn2_bias),
            (L2b3_conv1_w, L2b3_bn1_scale, L2b3_bn1_bias,
             L2b3_conv2_w, L2b3_bn2_scale, L2b3_bn2_bias),
            (L2b4_conv1_w, L2b4_bn1_scale, L2b4_bn1_bias,
             L2b4_conv2_w, L2b4_bn2_scale, L2b4_bn2_bias),
            (L2b5_conv1_w, L2b5_bn1_scale, L2b5_bn1_bias,
             L2b5_conv2_w, L2b5_bn2_scale, L2b5_bn2_bias)):
        xp = _fused_block(xp, w1, s1, b1, w2, s2, b2, 14, 14, 256)

    xp = _down_block(xp, 14, 14, 256, 512,
                     L3b0_conv1_w, L3b0_bn1_scale, L3b0_bn1_bias,
                     L3b0_conv2_w, L3b0_bn2_scale, L3b0_bn2_bias,
                     L3b0_down_w, L3b0_down_scale, L3b0_down_bias)
    for (w1, s1, b1, w2, s2, b2) in (
            (L3b1_conv1_w, L3b1_bn1_scale, L3b1_bn1_bias,
             L3b1_conv2_w, L3b1_bn2_scale, L3b1_bn2_bias),
            (L3b2_conv1_w, L3b2_bn1_scale, L3b2_bn1_bias,
             L3b2_conv2_w, L3b2_bn2_scale, L3b2_bn2_bias)):
        xp = _fused_block(xp, w1, s1, b1, w2, s2, b2, 7, 7, 512)

    L3 = 9 * 9 + 2
    n_out = fc_w.shape[1]
    logits = pl.pallas_call(
        functools.partial(_head_kernel, inv_hw=1.0 / 49.0),
        out_shape=jax.ShapeDtypeStruct((N, n_out), jnp.float32),
        grid=(1,),
        in_specs=[pl.BlockSpec((N, L3, 512), lambda i: (0, 0, 0)),
                  pl.BlockSpec((512, n_out), lambda i: (0, 0)),
                  pl.BlockSpec((1, n_out), lambda i: (0, 0))],
        out_specs=pl.BlockSpec((N, n_out), lambda i: (0, 0)),
        compiler_params=pltpu.CompilerParams(
            dimension_semantics=("arbitrary",),
            vmem_limit_bytes=_VMEM),
    )(xp, fc_w, fc_b)
    return logits[:, :1000]


# speedup vs baseline: 1.6301x; 1.4761x over previous
import functools
import itertools

import jax
import jax.numpy as jnp
from jax.experimental import pallas as pl
from jax.experimental.pallas import tpu as pltpu

_DT = jnp.bfloat16
_VMEM = 48 * 1024 * 1024
_TAPS3 = tuple(itertools.product(range(3), range(3)))


def _mask_cols(y, Wp, W):
    col = jax.lax.broadcasted_iota(jnp.int32, y.shape, 0) % Wp
    return jnp.where(col < W, y, 0.0)


def _fused_block_kernel(x_ref, w1_ref, s1_ref, b1_ref, w2_ref, s2_ref, b2_ref,
                        o_ref, acc_ref, mid_ref, *, H, W):
    Wp = W + 2
    l_out = H * Wp
    off0 = Wp + 1
    C = x_ref.shape[1]
    zrow = jnp.zeros((off0, C), _DT)
    mid_ref[0:off0, :] = zrow
    mid_ref[off0 + l_out:, :] = zrow
    for t, (di, dj) in enumerate(_TAPS3):
        off = di * Wp + dj
        c = jnp.dot(x_ref[off:off + l_out, :], w1_ref[t],
                    preferred_element_type=jnp.float32)
        if t == 0:
            acc_ref[...] = c
        else:
            acc_ref[...] += c
    y = jnp.maximum(acc_ref[...] * s1_ref[...] + b1_ref[...], 0.0)
    y = _mask_cols(y, Wp, W)
    mid_ref[off0:off0 + l_out, :] = y.astype(_DT)
    for t, (di, dj) in enumerate(_TAPS3):
        off = di * Wp + dj
        c = jnp.dot(mid_ref[off:off + l_out, :], w2_ref[t],
                    preferred_element_type=jnp.float32)
        if t == 0:
            acc_ref[...] = c
        else:
            acc_ref[...] += c
    y = acc_ref[...] * s2_ref[...] + b2_ref[...] \
        + x_ref[off0:off0 + l_out, :].astype(jnp.float32)
    y = jnp.maximum(y, 0.0)
    y = _mask_cols(y, Wp, W)
    o_ref[0:off0, :] = zrow
    o_ref[off0:off0 + l_out, :] = y.astype(_DT)
    o_ref[off0 + l_out:, :] = zrow


def _fused_block(xpad, w1, s1, b1, w2, s2, b2, H, W, C):
    N = xpad.shape[0]
    L = (H + 2) * (W + 2) + 2
    l_out = H * (W + 2)
    return pl.pallas_call(
        functools.partial(_fused_block_kernel, H=H, W=W),
        out_shape=jax.ShapeDtypeStruct((N, L, C), _DT),
        grid=(N,),
        in_specs=[
            pl.BlockSpec((None, L, C), lambda n: (n, 0, 0)),
            pl.BlockSpec((9, C, C), lambda n: (0, 0, 0)),
            pl.BlockSpec((1, C), lambda n: (0, 0)),
            pl.BlockSpec((1, C), lambda n: (0, 0)),
            pl.BlockSpec((9, C, C), lambda n: (0, 0, 0)),
            pl.BlockSpec((1, C), lambda n: (0, 0)),
            pl.BlockSpec((1, C), lambda n: (0, 0)),
        ],
        out_specs=pl.BlockSpec((None, L, C), lambda n: (n, 0, 0)),
        scratch_shapes=[pltpu.VMEM((l_out, C), jnp.float32),
                        pltpu.VMEM((L, C), _DT)],
        compiler_params=pltpu.CompilerParams(
            dimension_semantics=("parallel",),
            vmem_limit_bytes=_VMEM),
    )(xpad, w1, s1, b1, w2, s2, b2)


def _down_pair_kernel(x_ref, w1_ref, s1_ref, b1_ref, wd_ref, sd_ref, bd_ref,
                      y_ref, r_ref, acc_ref, *, taps, l_out):
    for t, (p, off) in enumerate(taps):
        c = jnp.dot(x_ref[p, off:off + l_out, :], w1_ref[t],
                    preferred_element_type=jnp.float32)
        if t == 0:
            acc_ref[...] = c
        else:
            acc_ref[...] += c
    y = jnp.maximum(acc_ref[...] * s1_ref[...] + b1_ref[...], 0.0)
    y_ref[...] = y.astype(_DT)
    rd = jnp.dot(x_ref[3, 0:l_out, :], wd_ref[0],
                 preferred_element_type=jnp.float32)
    r_ref[...] = (rd * sd_ref[...] + bd_ref[...]).astype(_DT)


def _conv_res_pad_kernel(x_ref, w_ref, s_ref, b_ref, r_ref, o_ref, acc_ref,
                         *, H, W):
    Wp = W + 2
    l_out = H * Wp
    off0 = Wp + 1
    C = o_ref.shape[1]
    for t, (di, dj) in enumerate(_TAPS3):
        off = di * Wp + dj
        c = jnp.dot(x_ref[off:off + l_out, :], w_ref[t],
                    preferred_element_type=jnp.float32)
        if t == 0:
            acc_ref[...] = c
        else:
            acc_ref[...] += c
    y = acc_ref[...] * s_ref[...] + b_ref[...] + r_ref[...].astype(jnp.float32)
    y = jnp.maximum(y, 0.0)
    y = _mask_cols(y, Wp, W)
    zrow = jnp.zeros((off0, C), _DT)
    o_ref[0:off0, :] = zrow
    o_ref[off0:off0 + l_out, :] = y.astype(_DT)
    o_ref[off0 + l_out:, :] = zrow


def _to_padded(x):
    N, H, W, C = x.shape
    xp = jnp.pad(x, ((0, 0), (1, 1), (1, 1), (0, 0)))
    return jnp.pad(xp.reshape(N, (H + 2) * (W + 2), C), ((0, 0), (0, 2), (0, 0)))


def _down_block(xpad, H, W, Cin, Cout, w1, s1, b1, w2, s2, b2, wd, sd, bd):
    N = xpad.shape[0]
    Wp = W + 2
    OH, OW = H // 2, W // 2
    PH, PW = OH + 1, OW + 1
    l_in = PH * PW + 1
    l_out1 = OH * PW
    xp = xpad[:, :(H + 2) * Wp, :].reshape(N, H + 2, Wp, Cin)
    phs = []
    for a in (0, 1):
        for b_ in (0, 1):
            sub = xp[:, a::2, b_::2, :][:, :PH, :PW, :]
            phs.append(sub.reshape(N, PH * PW, Cin))
    ph = jnp.stack(phs, 0)
    ph = jnp.pad(ph, ((0, 0), (0, 0), (0, l_in - PH * PW), (0, 0)))
    taps = [((ki % 2) * 2 + (kj % 2), (ki // 2) * PW + (kj // 2))
            for ki, kj in _TAPS3]
    y1, r = pl.pallas_call(
        functools.partial(_down_pair_kernel, taps=taps, l_out=l_out1),
        out_shape=[jax.ShapeDtypeStruct((N, l_out1, Cout), _DT),
                   jax.ShapeDtypeStruct((N, l_out1, Cout), _DT)],
        grid=(N,),
        in_specs=[
            pl.BlockSpec((4, None, l_in, Cin), lambda n: (0, n, 0, 0)),
            pl.BlockSpec((9, Cin, Cout), lambda n: (0, 0, 0)),
            pl.BlockSpec((1, Cout), lambda n: (0, 0)),
            pl.BlockSpec((1, Cout), lambda n: (0, 0)),
            pl.BlockSpec((1, Cin, Cout), lambda n: (0, 0, 0)),
            pl.BlockSpec((1, Cout), lambda n: (0, 0)),
            pl.BlockSpec((1, Cout), lambda n: (0, 0)),
        ],
        out_specs=[pl.BlockSpec((None, l_out1, Cout), lambda n: (n, 0, 0)),
                   pl.BlockSpec((None, l_out1, Cout), lambda n: (n, 0, 0))],
        scratch_shapes=[pltpu.VMEM((l_out1, Cout), jnp.float32)],
        compiler_params=pltpu.CompilerParams(
            dimension_semantics=("parallel",),
            vmem_limit_bytes=_VMEM),
    )(ph, w1, s1, b1, wd, sd, bd)
    y1p = _to_padded(y1.reshape(N, OH, PW, Cout)[:, :, :OW, :])
    rr = jnp.pad(r.reshape(N, OH, PW, Cout)[:, :, :OW, :],
                 ((0, 0), (0, 0), (0, 2), (0, 0)))
    rr = rr.reshape(N, OH * (OW + 2), Cout)
    L2 = (OH + 2) * (OW + 2) + 2
    l_out2 = OH * (OW + 2)
    return pl.pallas_call(
        functools.partial(_conv_res_pad_kernel, H=OH, W=OW),
        out_shape=jax.ShapeDtypeStruct((N, L2, Cout), _DT),
        grid=(N,),
        in_specs=[
            pl.BlockSpec((None, L2, Cout), lambda n: (n, 0, 0)),
            pl.BlockSpec((9, Cout, Cout), lambda n: (0, 0, 0)),
            pl.BlockSpec((1, Cout), lambda n: (0, 0)),
            pl.BlockSpec((1, Cout), lambda n: (0, 0)),
            pl.BlockSpec((None, l_out2, Cout), lambda n: (n, 0, 0)),
        ],
        out_specs=pl.BlockSpec((None, L2, Cout), lambda n: (n, 0, 0)),
        scratch_shapes=[pltpu.VMEM((l_out2, Cout), jnp.float32)],
        compiler_params=pltpu.CompilerParams(
            dimension_semantics=("parallel",),
            vmem_limit_bytes=_VMEM),
    )(y1p, w2, s2, b2, rr)


def _stem_kernel(x_ref, w_ref, s_ref, b_ref, o_ref):
    y = jnp.dot(x_ref[...], w_ref[...], preferred_element_type=jnp.float32)
    o_ref[...] = (y * s_ref[...] + b_ref[...]).astype(_DT)


def _maxpool_kernel(x_ref, o_ref, *, taps, l_out):
    p0, off0 = taps[0]
    acc = x_ref[p0, off0:off0 + l_out, :]
    for (p, off) in taps[1:]:
        acc = jnp.maximum(acc, x_ref[p, off:off + l_out, :])
    o_ref[...] = acc


def _head_kernel(x_ref, w_ref, b_ref, o_ref, *, inv_hw):
    pooled = jnp.sum(x_ref[...].astype(jnp.float32), axis=1) * inv_hw
    y = jnp.dot(pooled.astype(_DT), w_ref[...],
                preferred_element_type=jnp.float32)
    o_ref[...] = y + b_ref[...]


def _build_phases_s2(x, pad, PH, PW, l_in, pad_value):
    N, H, W, C = x.shape
    xp = jnp.pad(x, ((0, 0), (pad, pad), (pad, pad), (0, 0)),
                 constant_values=pad_value)
    phs = []
    for a in (0, 1):
        for b in (0, 1):
            sub = xp[:, a::2, b::2, :][:, :PH, :PW, :]
            ph_pad = PH - sub.shape[1]
            pw_pad = PW - sub.shape[2]
            if ph_pad or pw_pad:
                sub = jnp.pad(sub, ((0, 0), (0, ph_pad), (0, pw_pad), (0, 0)),
                              constant_values=pad_value)
            phs.append(sub.reshape(N, PH * PW, C))
    st = jnp.stack(phs, 0)
    return jnp.pad(st, ((0, 0), (0, 0), (0, l_in - PH * PW), (0, 0)),
                   constant_values=pad_value)


def kernel(x, conv1_w, bn1_scale, bn1_bias, L0b0_conv1_w, L0b0_bn1_scale, L0b0_bn1_bias, L0b0_conv2_w, L0b0_bn2_scale, L0b0_bn2_bias, L0b1_conv1_w, L0b1_bn1_scale, L0b1_bn1_bias, L0b1_conv2_w, L0b1_bn2_scale, L0b1_bn2_bias, L0b2_conv1_w, L0b2_bn1_scale, L0b2_bn1_bias, L0b2_conv2_w, L0b2_bn2_scale, L0b2_bn2_bias, L1b0_conv1_w, L1b0_bn1_scale, L1b0_bn1_bias, L1b0_conv2_w, L1b0_bn2_scale, L1b0_bn2_bias, L1b0_down_w, L1b0_down_scale, L1b0_down_bias, L1b1_conv1_w, L1b1_bn1_scale, L1b1_bn1_bias, L1b1_conv2_w, L1b1_bn2_scale, L1b1_bn2_bias, L1b2_conv1_w, L1b2_bn1_scale, L1b2_bn1_bias, L1b2_conv2_w, L1b2_bn2_scale, L1b2_bn2_bias, L1b3_conv1_w, L1b3_bn1_scale, L1b3_bn1_bias, L1b3_conv2_w, L1b3_bn2_scale, L1b3_bn2_bias, L2b0_conv1_w, L2b0_bn1_scale, L2b0_bn1_bias, L2b0_conv2_w, L2b0_bn2_scale, L2b0_bn2_bias, L2b0_down_w, L2b0_down_scale, L2b0_down_bias, L2b1_conv1_w, L2b1_bn1_scale, L2b1_bn1_bias, L2b1_conv2_w, L2b1_bn2_scale, L2b1_bn2_bias, L2b2_conv1_w, L2b2_bn1_scale, L2b2_bn1_bias, L2b2_conv2_w, L2b2_bn2_scale, L2b2_bn2_bias, L2b3_conv1_w, L2b3_bn1_scale, L2b3_bn1_bias, L2b3_conv2_w, L2b3_bn2_scale, L2b3_bn2_bias, L2b4_conv1_w, L2b4_bn1_scale, L2b4_bn1_bias, L2b4_conv2_w, L2b4_bn2_scale, L2b4_bn2_bias, L2b5_conv1_w, L2b5_bn1_scale, L2b5_bn1_bias, L2b5_conv2_w, L2b5_bn2_scale, L2b5_bn2_bias, L3b0_conv1_w, L3b0_bn1_scale, L3b0_bn1_bias, L3b0_conv2_w, L3b0_bn2_scale, L3b0_bn2_bias, L3b0_down_w, L3b0_down_scale, L3b0_down_bias, L3b1_conv1_w, L3b1_bn1_scale, L3b1_bn1_bias, L3b1_conv2_w, L3b1_bn2_scale, L3b1_bn2_bias, L3b2_conv1_w, L3b2_bn1_scale, L3b2_bn1_bias, L3b2_conv2_w, L3b2_bn2_scale, L3b2_bn2_bias, fc_w, fc_b):
    N = x.shape[0]
    xh = jnp.transpose(x, (0, 2, 3, 1)).astype(_DT)

    xpd = jnp.pad(xh, ((0, 0), (3, 3), (3, 3), (0, 0)))
    phg = [[xpd[:, a::2, b::2, :] for b in (0, 1)] for a in (0, 1)]
    cols = jnp.concatenate(
        [phg[ki % 2][kj % 2][:, ki // 2:ki // 2 + 112, kj // 2:kj // 2 + 112, :]
         for ki, kj in itertools.product(range(7), range(7))], axis=-1)
    cols = cols.reshape(N, 112 * 112, 147)
    w147 = conv1_w.reshape(147, 64)
    y = pl.pallas_call(
        _stem_kernel,
        out_shape=jax.ShapeDtypeStruct((N, 112 * 112, 64), _DT),
        grid=(N,),
        in_specs=[
            pl.BlockSpec((None, 112 * 112, 147), lambda n: (n, 0, 0)),
            pl.BlockSpec((147, 64), lambda n: (0, 0)),
            pl.BlockSpec((1, 64), lambda n: (0, 0)),
            pl.BlockSpec((1, 64), lambda n: (0, 0)),
        ],
        out_specs=pl.BlockSpec((None, 112 * 112, 64), lambda n: (n, 0, 0)),
        compiler_params=pltpu.CompilerParams(
            dimension_semantics=("parallel",),
            vmem_limit_bytes=_VMEM),
    )(cols, w147, bn1_scale, bn1_bias)
    y = y.reshape(N, 112, 112, 64)

    l_in = 57 * 57 + 1
    l_out = 56 * 57
    ph = _build_phases_s2(y, 1, 57, 57, l_in, -jnp.inf)
    taps = [((ki % 2) * 2 + (kj % 2), (ki // 2) * 57 + (kj // 2))
            for ki, kj in _TAPS3]
    y = pl.pallas_call(
        functools.partial(_maxpool_kernel, taps=taps, l_out=l_out),
        out_shape=jax.ShapeDtypeStruct((N, l_out, 64), _DT),
        grid=(N,),
        in_specs=[pl.BlockSpec((4, None, l_in, 64), lambda n: (0, n, 0, 0))],
        out_specs=pl.BlockSpec((None, l_out, 64), lambda n: (n, 0, 0)),
        compiler_params=pltpu.CompilerParams(
            dimension_semantics=("parallel",),
            vmem_limit_bytes=_VMEM),
    )(ph)
    y = y.reshape(N, 56, 57, 64)[:, :, :56, :]

    xp = _to_padded(y)

    for (w1, s1, b1, w2, s2, b2) in (
            (L0b0_conv1_w, L0b0_bn1_scale, L0b0_bn1_bias,
             L0b0_conv2_w, L0b0_bn2_scale, L0b0_bn2_bias),
            (L0b1_conv1_w, L0b1_bn1_scale, L0b1_bn1_bias,
             L0b1_conv2_w, L0b1_bn2_scale, L0b1_bn2_bias),
            (L0b2_conv1_w, L0b2_bn1_scale, L0b2_bn1_bias,
             L0b2_conv2_w, L0b2_bn2_scale, L0b2_bn2_bias)):
        xp = _fused_block(xp, w1, s1, b1, w2, s2, b2, 56, 56, 64)

    xp = _down_block(xp, 56, 56, 64, 128,
                     L1b0_conv1_w, L1b0_bn1_scale, L1b0_bn1_bias,
                     L1b0_conv2_w, L1b0_bn2_scale, L1b0_bn2_bias,
                     L1b0_down_w, L1b0_down_scale, L1b0_down_bias)
    for (w1, s1, b1, w2, s2, b2) in (
            (L1b1_conv1_w, L1b1_bn1_scale, L1b1_bn1_bias,
             L1b1_conv2_w, L1b1_bn2_scale, L1b1_bn2_bias),
            (L1b2_conv1_w, L1b2_bn1_scale, L1b2_bn1_bias,
             L1b2_conv2_w, L1b2_bn2_scale, L1b2_bn2_bias),
            (L1b3_conv1_w, L1b3_bn1_scale, L1b3_bn1_bias,
             L1b3_conv2_w, L1b3_bn2_scale, L1b3_bn2_bias)):
        xp = _fused_block(xp, w1, s1, b1, w2, s2, b2, 28, 28, 128)

    xp = _down_block(xp, 28, 28, 128, 256,
                     L2b0_conv1_w, L2b0_bn1_scale, L2b0_bn1_bias,
                     L2b0_conv2_w, L2b0_bn2_scale, L2b0_bn2_bias,
                     L2b0_down_w, L2b0_down_scale, L2b0_down_bias)
    for (w1, s1, b1, w2, s2, b2) in (
            (L2b1_conv1_w, L2b1_bn1_scale, L2b1_bn1_bias,
             L2b1_conv2_w, L2b1_bn2_scale, L2b1_bn2_bias),
            (L2b2_conv1_w, L2b2_bn1_scale, L2b2_bn1_bias,
             L2b2_conv2_w, L2b2_bn2_scale, L2b2_bn2_bias),
            (L2b3_conv1_w, L2b3_bn1_scale, L2b3_bn1_bias,
             L2b3_conv2_w, L2b3_bn2_scale, L2b3_bn2_bias),
            (L2b4_conv1_w, L2b4_bn1_scale, L2b4_bn1_bias,
             L2b4_conv2_w, L2b4_bn2_scale, L2b4_bn2_bias),
            (L2b5_conv1_w, L2b5_bn1_scale, L2b5_bn1_bias,
             L2b5_conv2_w, L2b5_bn2_scale, L2b5_bn2_bias)):
        xp = _fused_block(xp, w1, s1, b1, w2, s2, b2, 14, 14, 256)

    xp = _down_block(xp, 14, 14, 256, 512,
                     L3b0_conv1_w, L3b0_bn1_scale, L3b0_bn1_bias,
                     L3b0_conv2_w, L3b0_bn2_scale, L3b0_bn2_bias,
                     L3b0_down_w, L3b0_down_scale, L3b0_down_bias)
    for (w1, s1, b1, w2, s2, b2) in (
            (L3b1_conv1_w, L3b1_bn1_scale, L3b1_bn1_bias,
             L3b1_conv2_w, L3b1_bn2_scale, L3b1_bn2_bias),
            (L3b2_conv1_w, L3b2_bn1_scale, L3b2_bn1_bias,
             L3b2_conv2_w, L3b2_bn2_scale, L3b2_bn2_bias)):
        xp = _fused_block(xp, w1, s1, b1, w2, s2, b2, 7, 7, 512)

    L3 = 9 * 9 + 2
    n_out = fc_w.shape[1]
    logits = pl.pallas_call(
        functools.partial(_head_kernel, inv_hw=1.0 / 49.0),
        out_shape=jax.ShapeDtypeStruct((N, n_out), jnp.float32),
        grid=(1,),
        in_specs=[pl.BlockSpec((N, L3, 512), lambda i: (0, 0, 0)),
                  pl.BlockSpec((512, n_out), lambda i: (0, 0)),
                  pl.BlockSpec((1, n_out), lambda i: (0, 0))],
        out_specs=pl.BlockSpec((N, n_out), lambda i: (0, 0)),
        compiler_params=pltpu.CompilerParams(
            dimension_semantics=("arbitrary",),
            vmem_limit_bytes=_VMEM),
    )(xp, fc_w, fc_b)
    return logits[:, :1000]
